# baseline (speedup 1.0000x reference)
import functools

import jax
import jax.numpy as jnp
from jax import lax
from jax.experimental import pallas as pl
from jax.experimental.pallas import tpu as pltpu
from jax.experimental.pallas import tpu_sc as plsc

NUM_CLASSES = 10000
CLASSES_PER_TASK = 1000
CURRENT_TASK = 1
BAND_START = CURRENT_TASK * CLASSES_PER_TASK
BAND_END = BAND_START + CLASSES_PER_TASK

LANES = 16
HULL0 = (BAND_START // LANES) * LANES
N_FULL = (BAND_END - (HULL0 + LANES)) // LANES

ROWS = 4096
N_WORKERS = 32
ROWS_PER_WORKER = ROWS // N_WORKERS
SLOTS = 4


def _sc_body(x_hbm, alpha_hbm, beta_hbm, o_hbm, buf, ab_v, in_sem, out_sem):
    wid = lax.axis_index("s") * 2 + lax.axis_index("c")
    base = wid * ROWS_PER_WORKER

    def in_dma(k, slot):
        return pltpu.make_async_copy(
            x_hbm.at[pl.ds(base + k, 1), :],
            buf.at[slot],
            in_sem.at[slot],
        )

    def out_dma(k, slot):
        return pltpu.make_async_copy(
            buf.at[slot],
            o_hbm.at[pl.ds(base + k, 1), :],
            out_sem.at[slot],
        )

    pltpu.sync_copy(alpha_hbm, ab_v.at[pl.ds(0, 1)])
    pltpu.sync_copy(beta_hbm, ab_v.at[pl.ds(8, 1)])
    ab = ab_v[...]
    a = ab[0]
    b = ab[8]
    edge_mask = lax.iota(jnp.int32, LANES) >= (BAND_START - HULL0)

    for d in range(SLOTS - 1):
        in_dma(d, d).start()

    def correct_row(slot):
        v = buf[slot, 0, pl.ds(HULL0, LANES)]
        buf[slot, 0, pl.ds(HULL0, LANES)] = jnp.where(edge_mask, v * a + b, v)

        def fma(j, _):
            off = (HULL0 + LANES) + j * LANES
            buf[slot, 0, pl.ds(off, LANES)] = (
                buf[slot, 0, pl.ds(off, LANES)] * a + b)
            return 0

        lax.fori_loop(0, N_FULL, fma, 0)

    def outer(k0, _):
        for d in range(SLOTS):
            k = k0 + d
            nslot = (d + SLOTS - 1) % SLOTS

            @pl.when(k + SLOTS - 1 < ROWS_PER_WORKER)
            def _():
                @pl.when(k >= 1)
                def _():
                    out_dma(k - 1, nslot).wait()

                in_dma(k + SLOTS - 1, nslot).start()

            in_dma(k, d).wait()
            correct_row(d)
            out_dma(k, d).start()
        return 0

    lax.fori_loop(0, ROWS_PER_WORKER // SLOTS, lambda i, c: outer(i * SLOTS, c), 0)

    for k in range(ROWS_PER_WORKER - SLOTS, ROWS_PER_WORKER):
        out_dma(k, k % SLOTS).wait()


def kernel(x, alpha, beta):
    m, n = x.shape
    mesh = plsc.VectorSubcoreMesh(core_axis_name="c", subcore_axis_name="s")
    sc_kernel = functools.partial(
        pl.kernel,
        mesh=mesh,
        out_type=jax.ShapeDtypeStruct((m, n), x.dtype),
        scratch_types=[
            pltpu.VMEM((SLOTS, 1, NUM_CLASSES), jnp.float32),
            pltpu.VMEM((16,), jnp.float32),
            pltpu.SemaphoreType.DMA((SLOTS,)),
            pltpu.SemaphoreType.DMA((SLOTS,)),
        ],
    )(_sc_body)
    return sc_kernel(x, alpha, beta)
